# Initial kernel scaffold; baseline (speedup 1.0000x reference)
import functools

import jax
import jax.numpy as jnp
from jax import lax
from jax.experimental import pallas as pl
from jax.experimental.pallas import tpu as pltpu
from jax.experimental.pallas import tpu_sc as plsc

NC = 2
NS = 16
NW = NC * NS
L = 16
CHUNK = 128


def _mm_body(x_ref, w_ref, o_ref):
    o_ref[...] = jnp.dot(x_ref[...], w_ref[...],
                         preferred_element_type=jnp.float32)


def _fin_body(p_ref, n_ref, o_ref):
    s = p_ref[0] + p_ref[1]
    o_ref[...] = jnp.maximum(s * n_ref[...], 0.0)


@functools.lru_cache(maxsize=None)
def _make_agg(n, d, nchunk):
    rpt = n // NS
    fb = d // L
    mesh = plsc.VectorSubcoreMesh(core_axis_name="c", subcore_axis_name="s")

    @functools.partial(
        pl.kernel,
        out_type=jax.ShapeDtypeStruct((NC, n, d), jnp.float32),
        mesh=mesh,
        scratch_types=[
            pltpu.VMEM((nchunk, CHUNK), jnp.int32),
            pltpu.VMEM((nchunk, CHUNK), jnp.int32),
            pltpu.VMEM((nchunk, CHUNK), jnp.float32),
            pltpu.VMEM((CHUNK, d), jnp.float32),
            pltpu.VMEM_SHARED((n, d), jnp.float32),
            pltpu.SemaphoreType.DMA,
        ],
    )
    def agg(prod, srcs, dsts, ws, zrows, out, src_v, dst_v, w_v, rows_v,
            acc, sem):
        cid = lax.axis_index("c")
        sid = lax.axis_index("s")
        wid = sid * NC + cid

        pltpu.sync_copy(srcs.at[wid], src_v)
        pltpu.sync_copy(dsts.at[wid], dst_v)
        pltpu.sync_copy(ws.at[wid], w_v)
        pltpu.sync_copy(zrows, acc.at[pl.ds(sid * rpt, rpt)])
        plsc.subcore_barrier()

        def chunk_body(c, carry):
            pltpu.async_copy(prod.at[src_v.at[c]], rows_v, sem).wait()

            def grp(g, carry2):
                w16 = w_v[c, pl.ds(g * L, L)]
                for j in range(L):
                    wj = jnp.take(w16, jnp.full((L,), j, jnp.int32),
                                  mode="promise_in_bounds")
                    e = g * L + j
                    for f in range(fb):
                        blk = rows_v[e, pl.ds(f * L, L)]
                        rows_v[e, pl.ds(f * L, L)] = blk * wj
                return carry2

            lax.fori_loop(0, CHUNK // L, grp, 0)
            pltpu.sync_copy(rows_v, acc.at[dst_v.at[c]], add=True)
            return carry

        lax.fori_loop(0, nchunk, chunk_body, 0)
        plsc.subcore_barrier()
        pltpu.sync_copy(acc.at[pl.ds(sid * rpt, rpt)],
                        out.at[cid, pl.ds(sid * rpt, rpt)])

    return agg


def kernel(x, edge_index, edge_weight, n_norm, W):
    n, _ = x.shape
    d = W.shape[1]
    e = edge_weight.shape[0]

    nchunk = -(-e // (NW * CHUNK))
    epad = NW * nchunk * CHUNK
    pad = epad - e
    src = edge_index[0]
    dst = edge_index[1]
    ew = edge_weight
    if pad:
        src = jnp.concatenate([src, jnp.zeros((pad,), jnp.int32)])
        dst = jnp.concatenate(
            [dst, jnp.arange(pad, dtype=jnp.int32) % jnp.int32(n)])
        ew = jnp.concatenate([ew, jnp.zeros((pad,), jnp.float32)])
    src3 = src.reshape(NW, nchunk, CHUNK)
    dst3 = dst.reshape(NW, nchunk, CHUNK)
    w3 = ew.reshape(NW, nchunk, CHUNK)
    zrows = jnp.zeros((n // NS, d), jnp.float32)

    rows_blk = 400 if n % 400 == 0 else n
    grid = n // rows_blk
    prod = pl.pallas_call(
        _mm_body,
        grid=(grid,),
        in_specs=[
            pl.BlockSpec((rows_blk, x.shape[1]), lambda i: (i, 0)),
            pl.BlockSpec((x.shape[1], d), lambda i: (0, 0)),
        ],
        out_specs=pl.BlockSpec((rows_blk, d), lambda i: (i, 0)),
        out_shape=jax.ShapeDtypeStruct((n, d), jnp.float32),
    )(x, W)

    partials = _make_agg(n, d, nchunk)(prod, src3, dst3, w3, zrows)

    out = pl.pallas_call(
        _fin_body,
        grid=(grid,),
        in_specs=[
            pl.BlockSpec((NC, rows_blk, d), lambda i: (0, i, 0)),
            pl.BlockSpec((rows_blk, 1), lambda i: (i, 0)),
        ],
        out_specs=pl.BlockSpec((rows_blk, d), lambda i: (i, 0)),
        out_shape=jax.ShapeDtypeStruct((n, d), jnp.float32),
    )(partials, n_norm)
    return out


# speedup vs baseline: 4.6153x; 4.6153x over previous
import functools

import jax
import jax.numpy as jnp
from jax import lax
from jax.experimental import pallas as pl
from jax.experimental.pallas import tpu as pltpu
from jax.experimental.pallas import tpu_sc as plsc

NC = 2
NS = 16
NW = NC * NS
L = 16
CHUNK = 128

_GATHER_DNUMS = lax.GatherDimensionNumbers(
    offset_dims=(), collapsed_slice_dims=(0,), start_index_map=(0,))


def _mm_body(x_ref, w_ref, o_ref):
    o_ref[...] = jnp.dot(x_ref[...], w_ref[...],
                         preferred_element_type=jnp.float32)


def _fin_body(p_ref, n_ref, o_ref):
    s = p_ref[0] + p_ref[1]
    o_ref[...] = jnp.maximum(s * n_ref[...], 0.0)


def _rows_per_tile(n):
    rpt = -(-n // NS)
    return (rpt + 7) // 8 * 8


@functools.lru_cache(maxsize=None)
def _make_agg(n, d, nchunk):
    rpt = _rows_per_tile(n)
    npad = rpt * NS
    fb = d // L
    mesh = plsc.VectorSubcoreMesh(core_axis_name="c", subcore_axis_name="s")

    @functools.partial(
        pl.kernel,
        out_type=jax.ShapeDtypeStruct((NC, npad, d), jnp.float32),
        mesh=mesh,
        scratch_types=[
            pltpu.VMEM((nchunk, CHUNK), jnp.int32),
            pltpu.VMEM((nchunk, CHUNK), jnp.int32),
            pltpu.VMEM((nchunk, CHUNK), jnp.float32),
            pltpu.VMEM((CHUNK, d), jnp.float32),
            pltpu.VMEM_SHARED((npad, d), jnp.float32),
            pltpu.SemaphoreType.DMA,
        ],
    )
    def agg(prod, srcs, dsts, ws, zrows, out, src_v, dst_v, w_v, rows_v,
            acc, sem):
        cid = lax.axis_index("c")
        sid = lax.axis_index("s")
        wid = sid * NC + cid

        pltpu.sync_copy(srcs.at[wid], src_v)
        pltpu.sync_copy(dsts.at[wid], dst_v)
        pltpu.sync_copy(ws.at[wid], w_v)
        pltpu.sync_copy(zrows, acc.at[pl.ds(sid * rpt, rpt)])
        plsc.subcore_barrier()

        def chunk_body(c, carry):
            pltpu.async_copy(prod.at[src_v.at[c]], rows_v, sem).wait()

            def grp(g, carry2):
                w16 = w_v[c, pl.ds(g * L, L)]
                for j in range(L):
                    wj = lax.gather(
                        w16, jnp.full((L, 1), j, jnp.int32),
                        _GATHER_DNUMS, slice_sizes=(1,),
                        mode=lax.GatherScatterMode.PROMISE_IN_BOUNDS)
                    e = g * L + j
                    for f in range(fb):
                        blk = rows_v[e, pl.ds(f * L, L)]
                        rows_v[e, pl.ds(f * L, L)] = blk * wj
                return carry2

            lax.fori_loop(0, CHUNK // L, grp, 0)
            pltpu.sync_copy(rows_v, acc.at[dst_v.at[c]], add=True)
            return carry

        lax.fori_loop(0, nchunk, chunk_body, 0)
        plsc.subcore_barrier()
        pltpu.sync_copy(acc.at[pl.ds(sid * rpt, rpt)],
                        out.at[cid, pl.ds(sid * rpt, rpt)])

    return agg


def kernel(x, edge_index, edge_weight, n_norm, W):
    n, _ = x.shape
    d = W.shape[1]
    e = edge_weight.shape[0]

    nchunk = -(-e // (NW * CHUNK))
    epad = NW * nchunk * CHUNK
    pad = epad - e
    src = edge_index[0]
    dst = edge_index[1]
    ew = edge_weight
    if pad:
        src = jnp.concatenate([src, jnp.zeros((pad,), jnp.int32)])
        dst = jnp.concatenate(
            [dst, jnp.arange(pad, dtype=jnp.int32) % jnp.int32(n)])
        ew = jnp.concatenate([ew, jnp.zeros((pad,), jnp.float32)])
    src3 = src.reshape(NW, nchunk, CHUNK)
    dst3 = dst.reshape(NW, nchunk, CHUNK)
    w3 = ew.reshape(NW, nchunk, CHUNK)
    zrows = jnp.zeros((_rows_per_tile(n), d), jnp.float32)

    rows_blk = 400 if n % 400 == 0 else n
    grid = n // rows_blk
    prod = pl.pallas_call(
        _mm_body,
        grid=(grid,),
        in_specs=[
            pl.BlockSpec((rows_blk, x.shape[1]), lambda i: (i, 0)),
            pl.BlockSpec((x.shape[1], d), lambda i: (0, 0)),
        ],
        out_specs=pl.BlockSpec((rows_blk, d), lambda i: (i, 0)),
        out_shape=jax.ShapeDtypeStruct((n, d), jnp.float32),
    )(x, W)

    partials = _make_agg(n, d, nchunk)(prod, src3, dst3, w3, zrows)

    out = pl.pallas_call(
        _fin_body,
        grid=(grid,),
        in_specs=[
            pl.BlockSpec((NC, rows_blk, d), lambda i: (0, i, 0)),
            pl.BlockSpec((rows_blk, 1), lambda i: (i, 0)),
        ],
        out_specs=pl.BlockSpec((rows_blk, d), lambda i: (i, 0)),
        out_shape=jax.ShapeDtypeStruct((n, d), jnp.float32),
    )(partials, n_norm)
    return out
